# baseline (speedup 1.0000x reference)
import functools

import jax
import jax.numpy as jnp
from jax import lax
from jax.experimental import pallas as pl
from jax.experimental.pallas import tpu as pltpu
from jax.experimental.pallas import tpu_sc as plsc

NUM_COMPONENTS = 32
BATCH_U = 4096
BATCH_I = 4096

_info = plsc.get_sparse_core_info()
_NC = _info.num_cores
_NS = _info.num_subcores
_NW = _NC * _NS
_ROWS_PER_W = BATCH_U // _NW

_sc_mesh = plsc.VectorSubcoreMesh(core_axis_name="c", subcore_axis_name="s")


@functools.partial(
    pl.kernel,
    out_type=[
        jax.ShapeDtypeStruct((BATCH_U, NUM_COMPONENTS), jnp.float32),
        jax.ShapeDtypeStruct((BATCH_I, NUM_COMPONENTS), jnp.float32),
    ],
    mesh=_sc_mesh,
    scratch_types=[
        pltpu.VMEM((_ROWS_PER_W,), jnp.int32),
        pltpu.VMEM((_ROWS_PER_W, NUM_COMPONENTS), jnp.float32),
        pltpu.VMEM((_ROWS_PER_W,), jnp.int32),
        pltpu.VMEM((_ROWS_PER_W, NUM_COMPONENTS), jnp.float32),
        pltpu.SemaphoreType.DMA,
        pltpu.SemaphoreType.DMA,
    ],
    compiler_params=pltpu.CompilerParams(use_tc_tiling_on_sc=False),
)
def _sc_gather(user_idx_hbm, item_idx_hbm, user_emb_hbm, item_emb_hbm,
               wu_hbm, hi_hbm, uidx_v, urows_v, iidx_v, irows_v, usem, isem):
    wid = lax.axis_index("s") * _NC + lax.axis_index("c")
    base = wid * _ROWS_PER_W
    pltpu.sync_copy(user_idx_hbm.at[pl.ds(base, _ROWS_PER_W)], uidx_v)
    pltpu.sync_copy(item_idx_hbm.at[pl.ds(base, _ROWS_PER_W)], iidx_v)
    ucp = pltpu.async_copy(user_emb_hbm.at[uidx_v], urows_v, usem)
    icp = pltpu.async_copy(item_emb_hbm.at[iidx_v], irows_v, isem)
    ucp.wait()
    icp.wait()
    pltpu.sync_copy(urows_v, wu_hbm.at[pl.ds(base, _ROWS_PER_W)])
    pltpu.sync_copy(irows_v, hi_hbm.at[pl.ds(base, _ROWS_PER_W)])


_BLK_U = 512


def _mm_body(wu_ref, hi_ref, out_ref):
    out_ref[...] = lax.dot_general(
        wu_ref[...], hi_ref[...],
        dimension_numbers=(((1,), (1,)), ((), ())),
        preferred_element_type=jnp.float32,
    )


_matmul = pl.pallas_call(
    _mm_body,
    grid=(BATCH_U // _BLK_U,),
    in_specs=[
        pl.BlockSpec((_BLK_U, NUM_COMPONENTS), lambda i: (i, 0)),
        pl.BlockSpec((BATCH_I, NUM_COMPONENTS), lambda i: (0, 0)),
    ],
    out_specs=pl.BlockSpec((_BLK_U, BATCH_I), lambda i: (i, 0)),
    out_shape=jax.ShapeDtypeStruct((BATCH_U, BATCH_I), jnp.float32),
)


@jax.jit
def kernel(user_tensor, item_tensor, user_embedding, item_embedding):
    w_u, h_i = _sc_gather(user_tensor, item_tensor,
                          user_embedding, item_embedding)
    return _matmul(w_u, h_i)


# speedup vs baseline: 1.4974x; 1.4974x over previous
import functools

import jax
import jax.numpy as jnp
from jax import lax
from jax.experimental import pallas as pl
from jax.experimental.pallas import tpu as pltpu
from jax.experimental.pallas import tpu_sc as plsc

NUM_COMPONENTS = 32
BATCH_U = 4096
BATCH_I = 4096

_info = plsc.get_sparse_core_info()
_NC = _info.num_cores
_NS = _info.num_subcores
_NW = _NC * _NS
_ROWS_PER_W = BATCH_U // _NW

_sc_mesh = plsc.VectorSubcoreMesh(core_axis_name="c", subcore_axis_name="s")


@functools.partial(
    pl.kernel,
    out_type=[
        jax.ShapeDtypeStruct((BATCH_U, NUM_COMPONENTS), jnp.float32),
        jax.ShapeDtypeStruct((BATCH_I, NUM_COMPONENTS), jnp.float32),
    ],
    mesh=_sc_mesh,
    scratch_types=[
        pltpu.VMEM((_ROWS_PER_W,), jnp.int32),
        pltpu.VMEM((_ROWS_PER_W, NUM_COMPONENTS), jnp.float32),
        pltpu.VMEM((_ROWS_PER_W,), jnp.int32),
        pltpu.VMEM((_ROWS_PER_W, NUM_COMPONENTS), jnp.float32),
        pltpu.SemaphoreType.DMA,
        pltpu.SemaphoreType.DMA,
    ],
)
def _sc_gather(user_idx_hbm, item_idx_hbm, user_emb_hbm, item_emb_hbm,
               wu_hbm, hi_hbm, uidx_v, urows_v, iidx_v, irows_v,
               usem, isem):
    wid = lax.axis_index("s") * _NC + lax.axis_index("c")
    base = wid * _ROWS_PER_W
    pltpu.sync_copy(user_idx_hbm.at[pl.ds(base, _ROWS_PER_W)], uidx_v)
    pltpu.sync_copy(item_idx_hbm.at[pl.ds(base, _ROWS_PER_W)], iidx_v)

    def fire(v, carry):
        uvec = uidx_v[pl.ds(v * 16, 16)]
        ivec = iidx_v[pl.ds(v * 16, 16)]
        for lane in range(16):
            j = v * 16 + lane
            pltpu.async_copy(user_emb_hbm.at[pl.ds(uvec[lane], 1)],
                             urows_v.at[pl.ds(j, 1)], usem)
            pltpu.async_copy(item_emb_hbm.at[pl.ds(ivec[lane], 1)],
                             irows_v.at[pl.ds(j, 1)], isem)
        return carry

    lax.fori_loop(0, _ROWS_PER_W // 16, fire, 0)
    pltpu.make_async_copy(user_emb_hbm.at[pl.ds(0, _ROWS_PER_W)],
                          urows_v, usem).wait()
    pltpu.make_async_copy(item_emb_hbm.at[pl.ds(0, _ROWS_PER_W)],
                          irows_v, isem).wait()
    pltpu.sync_copy(urows_v, wu_hbm.at[pl.ds(base, _ROWS_PER_W)])
    pltpu.sync_copy(irows_v, hi_hbm.at[pl.ds(base, _ROWS_PER_W)])


_BLK_U = 512


def _mm_body(wu_ref, hi_ref, out_ref):
    out_ref[...] = lax.dot_general(
        wu_ref[...], hi_ref[...],
        dimension_numbers=(((1,), (1,)), ((), ())),
        preferred_element_type=jnp.float32,
    )


_matmul = pl.pallas_call(
    _mm_body,
    grid=(BATCH_U // _BLK_U,),
    in_specs=[
        pl.BlockSpec((_BLK_U, NUM_COMPONENTS), lambda i: (i, 0)),
        pl.BlockSpec((BATCH_I, NUM_COMPONENTS), lambda i: (0, 0)),
    ],
    out_specs=pl.BlockSpec((_BLK_U, BATCH_I), lambda i: (i, 0)),
    out_shape=jax.ShapeDtypeStruct((BATCH_U, BATCH_I), jnp.float32),
)


@jax.jit
def kernel(user_tensor, item_tensor, user_embedding, item_embedding):
    w_u, h_i = _sc_gather(user_tensor, item_tensor,
                          user_embedding, item_embedding)
    return _matmul(w_u, h_i)


# speedup vs baseline: 9.2404x; 6.1711x over previous
import functools

import jax
import jax.numpy as jnp
from jax import lax
from jax.experimental import pallas as pl
from jax.experimental.pallas import tpu as pltpu
from jax.experimental.pallas import tpu_sc as plsc

NUM_COMPONENTS = 32
BATCH_U = 4096
BATCH_I = 4096

_info = plsc.get_sparse_core_info()
_NC = _info.num_cores
_NS = _info.num_subcores
_NW = _NC * _NS
_IDS_PER_W = BATCH_U // _NW
_NBUF = 16
_NGRP = _IDS_PER_W // _NBUF

_sc_mesh = plsc.VectorSubcoreMesh(core_axis_name="c", subcore_axis_name="s")


@functools.partial(
    pl.kernel,
    out_type=[
        jax.ShapeDtypeStruct((NUM_COMPONENTS, BATCH_U), jnp.float32),
        jax.ShapeDtypeStruct((NUM_COMPONENTS, BATCH_I), jnp.float32),
    ],
    mesh=_sc_mesh,
    scratch_types=[
        pltpu.VMEM((_IDS_PER_W,), jnp.int32),
        pltpu.VMEM((_IDS_PER_W,), jnp.int32),
        pltpu.VMEM((_NBUF, NUM_COMPONENTS, 128), jnp.float32),
        pltpu.VMEM((NUM_COMPONENTS, _IDS_PER_W), jnp.float32),
        pltpu.VMEM((NUM_COMPONENTS, _IDS_PER_W), jnp.float32),
    ] + [pltpu.SemaphoreType.DMA] * _NBUF,
    compiler_params=pltpu.CompilerParams(needs_layout_passes=False),
)
def _sc_gather(user_idx_hbm, item_idx_hbm, user_emb_t_hbm, item_emb_t_hbm,
               wu_t_hbm, hi_t_hbm, uidx_v, iidx_v, ring_v, ucols_v, icols_v,
               *sems):
    wid = lax.axis_index("s") * _NC + lax.axis_index("c")
    base = wid * _IDS_PER_W
    pltpu.sync_copy(user_idx_hbm.at[pl.ds(base, _IDS_PER_W)], uidx_v)
    pltpu.sync_copy(item_idx_hbm.at[pl.ds(base, _IDS_PER_W)], iidx_v)

    c_iota = lax.iota(jnp.int32, 16)

    def run_table(emb_t_hbm, idx_v, cols_v):
        def fire(g, b):
            vec = idx_v[pl.ds(g * _NBUF, _NBUF)]
            rid = vec[b]
            toff = pl.multiple_of((rid // 128) * 128, 128)
            pltpu.async_copy(emb_t_hbm.at[:, pl.ds(toff, 128)],
                             ring_v.at[b], sems[b])

        def extract(g, b):
            vec = idx_v[pl.ds(g * _NBUF, _NBUF)]
            lane = jnp.broadcast_to(vec[b] % 128, (16,))
            col = jnp.broadcast_to(g * _NBUF + b, (16,))
            lo = plsc.load_gather(ring_v.at[b], [c_iota, lane])
            hi = plsc.load_gather(ring_v.at[b], [c_iota + 16, lane])
            plsc.store_scatter(cols_v, [c_iota, col], lo)
            plsc.store_scatter(cols_v, [c_iota + 16, col], hi)

        def group(g, carry):
            for b in range(_NBUF):
                @pl.when(g > 0)
                def _drain():
                    pltpu.make_async_copy(
                        emb_t_hbm.at[:, pl.ds(0, 128)], ring_v.at[b],
                        sems[b]).wait()
                    extract(g - 1, b)
                fire(g, b)
            return carry

        lax.fori_loop(0, _NGRP, group, 0)
        for b in range(_NBUF):
            pltpu.make_async_copy(emb_t_hbm.at[:, pl.ds(0, 128)],
                                  ring_v.at[b], sems[b]).wait()
            extract(_NGRP - 1, b)

    run_table(user_emb_t_hbm, uidx_v, ucols_v)
    run_table(item_emb_t_hbm, iidx_v, icols_v)
    pltpu.sync_copy(ucols_v, wu_t_hbm.at[:, pl.ds(base, _IDS_PER_W)])
    pltpu.sync_copy(icols_v, hi_t_hbm.at[:, pl.ds(base, _IDS_PER_W)])


_BLK_U = 512


def _mm_body(wu_t_ref, hi_t_ref, out_ref):
    out_ref[...] = lax.dot_general(
        wu_t_ref[...], hi_t_ref[...],
        dimension_numbers=(((0,), (0,)), ((), ())),
        preferred_element_type=jnp.float32,
    )


_matmul = pl.pallas_call(
    _mm_body,
    grid=(BATCH_U // _BLK_U,),
    in_specs=[
        pl.BlockSpec((NUM_COMPONENTS, _BLK_U), lambda i: (0, i)),
        pl.BlockSpec((NUM_COMPONENTS, BATCH_I), lambda i: (0, 0)),
    ],
    out_specs=pl.BlockSpec((_BLK_U, BATCH_I), lambda i: (i, 0)),
    out_shape=jax.ShapeDtypeStruct((BATCH_U, BATCH_I), jnp.float32),
)


@jax.jit
def kernel(user_tensor, item_tensor, user_embedding, item_embedding):
    ue_t = user_embedding.T
    ie_t = item_embedding.T
    wu_t, hi_t = _sc_gather(user_tensor, item_tensor, ue_t, ie_t)
    return _matmul(wu_t, hi_t)
